# baseline (speedup 1.0000x reference)
import functools

import jax
import jax.numpy as jnp
from jax import lax
from jax.experimental import pallas as pl
from jax.experimental.pallas import tpu as pltpu
from jax.experimental.pallas import tpu_sc as plsc

E = 320000
D = 128
N = 10000
NC = 2
NS = 16
EPT = E // (NC * NS)
SUBA = 128
SUBB = 64
MROWS = SUBA + SUBB
NBUF = 2
NFULL = EPT // MROWS
TAIL = EPT - NFULL * MROWS
ZROWS = 624
ZCHUNK = 104
ZTAIL = N - NS * ZROWS


def _sc_segment_sum(M, dest):
    mesh = plsc.VectorSubcoreMesh(core_axis_name="c", subcore_axis_name="s")

    @functools.partial(
        pl.kernel,
        out_type=jax.ShapeDtypeStruct((NC, N, D), jnp.float32),
        mesh=mesh,
        scratch_types=[
            pltpu.VMEM((MROWS, D), jnp.float32),
            pltpu.VMEM((MROWS, D), jnp.float32),
            pltpu.VMEM((SUBA,), jnp.int32),
            pltpu.VMEM((SUBA,), jnp.int32),
            pltpu.VMEM((SUBB,), jnp.int32),
            pltpu.VMEM((SUBB,), jnp.int32),
            pltpu.VMEM((TAIL,), jnp.int32),
            pltpu.VMEM_SHARED((N, D), jnp.float32),
            pltpu.SemaphoreType.DMA,
            pltpu.SemaphoreType.DMA,
            pltpu.SemaphoreType.DMA,
            pltpu.SemaphoreType.DMA,
        ],
    )
    def k(m_hbm, d_hbm, out_hbm, mb0, mb1, iba0, iba1, ibb0, ibb1, itail,
          acc, ms0, ms1, is0, is1):
        c = lax.axis_index("c")
        s = lax.axis_index("s")
        mbs = (mb0, mb1)
        ibas = (iba0, iba1)
        ibbs = (ibb0, ibb1)
        msems = (ms0, ms1)
        isems = (is0, is1)

        ebase = c * (NS * EPT) + s * EPT

        def start_in(j, b):
            off = ebase + j * MROWS
            pltpu.async_copy(d_hbm.at[pl.ds(off, SUBA)], ibas[b], isems[b])
            pltpu.async_copy(d_hbm.at[pl.ds(off + SUBA, SUBB)], ibbs[b],
                             isems[b])
            pltpu.async_copy(m_hbm.at[pl.ds(off, MROWS)], mbs[b], msems[b])

        def wait_in(b):
            pltpu.make_async_copy(
                d_hbm.at[pl.ds(0, SUBA)], ibas[b], isems[b]).wait()
            pltpu.make_async_copy(
                d_hbm.at[pl.ds(0, SUBB)], ibbs[b], isems[b]).wait()
            pltpu.make_async_copy(
                m_hbm.at[pl.ds(0, MROWS)], mbs[b], msems[b]).wait()

        start_in(0, 1)

        zero = jnp.zeros((16,), jnp.float32)

        @pl.loop(0, ZCHUNK)
        def _(r):
            @pl.loop(0, D, step=16)
            def _(col):
                mb0[r, pl.ds(col, 16)] = zero

        @pl.loop(0, ZROWS // ZCHUNK)
        def _(j):
            pltpu.async_copy(
                mb0.at[pl.ds(0, ZCHUNK)],
                acc.at[pl.ds(s * ZROWS + j * ZCHUNK, ZCHUNK)],
                ms0,
            )

        @pl.when(s == NS - 1)
        def _():
            pltpu.async_copy(
                mb0.at[pl.ds(0, ZTAIL)],
                acc.at[pl.ds(NS * ZROWS, ZTAIL)],
                ms0,
            )

        @pl.loop(0, ZROWS // ZCHUNK)
        def _(j):
            pltpu.make_async_copy(
                mb0.at[pl.ds(0, ZCHUNK)],
                acc.at[pl.ds(0, ZCHUNK)],
                ms0,
            ).wait()

        @pl.when(s == NS - 1)
        def _():
            pltpu.make_async_copy(
                mb0.at[pl.ds(0, ZTAIL)],
                acc.at[pl.ds(0, ZTAIL)],
                ms0,
            ).wait()

        plsc.subcore_barrier()

        start_in(1, 0)

        RB = (1, 0)

        @pl.loop(0, NFULL // NBUF)
        def _(it):
            j0 = it * NBUF
            hs = []
            for k in range(NBUF):
                b = RB[k]
                wait_in(b)
                hs.append(pltpu.async_copy(
                    mbs[b].at[pl.ds(0, SUBA)], acc.at[ibas[b]],
                    msems[b], add=True))
                hs.append(pltpu.async_copy(
                    mbs[b].at[pl.ds(SUBA, SUBB)], acc.at[ibbs[b]],
                    msems[b], add=True))
            for k in range(NBUF):
                hs[2 * k].wait()
                hs[2 * k + 1].wait()

                @pl.when(j0 + k + NBUF < NFULL)
                def _(k=k):
                    start_in(j0 + k + NBUF, RB[k])

        toff = ebase + NFULL * MROWS
        pltpu.sync_copy(d_hbm.at[pl.ds(toff, TAIL)], itail)
        pltpu.sync_copy(m_hbm.at[pl.ds(toff, TAIL)], mb0.at[pl.ds(0, TAIL)])
        pltpu.sync_copy(mb0.at[pl.ds(0, TAIL)], acc.at[itail], add=True)

        plsc.subcore_barrier()

        @pl.loop(0, ZROWS // ZCHUNK)
        def _(j):
            row = s * ZROWS + j * ZCHUNK
            pltpu.async_copy(
                acc.at[pl.ds(row, ZCHUNK)],
                out_hbm.at[c].at[pl.ds(row, ZCHUNK)],
                ms0,
            )

        @pl.when(s == NS - 1)
        def _():
            pltpu.async_copy(
                acc.at[pl.ds(NS * ZROWS, ZTAIL)],
                out_hbm.at[c].at[pl.ds(NS * ZROWS, ZTAIL)],
                ms0,
            )

        @pl.loop(0, ZROWS // ZCHUNK)
        def _(j):
            pltpu.make_async_copy(
                acc.at[pl.ds(0, ZCHUNK)],
                out_hbm.at[c].at[pl.ds(0, ZCHUNK)],
                ms0,
            ).wait()

        @pl.when(s == NS - 1)
        def _():
            pltpu.make_async_copy(
                acc.at[pl.ds(0, ZTAIL)],
                out_hbm.at[c].at[pl.ds(0, ZTAIL)],
                ms0,
            ).wait()

    return k(M, dest)


def _tc_add_kernel(a_ref, b_ref, o_ref):
    o_ref[...] = a_ref[0] + b_ref[0]


def _tc_add(partials):
    blk = 1000
    return pl.pallas_call(
        _tc_add_kernel,
        out_shape=jax.ShapeDtypeStruct((N, D), jnp.float32),
        grid=(N // blk,),
        in_specs=[
            pl.BlockSpec((1, blk, D), lambda i: (0, i, 0)),
            pl.BlockSpec((1, blk, D), lambda i: (1, i, 0)),
        ],
        out_specs=pl.BlockSpec((blk, D), lambda i: (i, 0)),
    )(partials, partials)


def kernel(M, dest, dim_size):
    partials = _sc_segment_sum(M, dest.astype(jnp.int32))
    out = _tc_add(partials)
    w = jnp.ones((E, 1), dtype=M.dtype)
    return (out, w)


# speedup vs baseline: 1.0816x; 1.0816x over previous
import functools

import jax
import jax.numpy as jnp
from jax import lax
from jax.experimental import pallas as pl
from jax.experimental.pallas import tpu as pltpu
from jax.experimental.pallas import tpu_sc as plsc

E = 320000
D = 128
N = 10000
NC = 2
NS = 16
EPT = E // (NC * NS)
CHUNK = 128
NBUF = 3
NFULL = EPT // CHUNK
TAIL = EPT - NFULL * CHUNK
ZROWS = 624
ZCHUNK = 104
ZTAIL = N - NS * ZROWS


def _sc_segment_sum(M, dest):
    mesh = plsc.VectorSubcoreMesh(core_axis_name="c", subcore_axis_name="s")

    @functools.partial(
        pl.kernel,
        out_type=jax.ShapeDtypeStruct((NC, N, D), jnp.float32),
        mesh=mesh,
        scratch_types=[
            pltpu.VMEM((CHUNK, D), jnp.float32),
            pltpu.VMEM((CHUNK, D), jnp.float32),
            pltpu.VMEM((CHUNK, D), jnp.float32),
            pltpu.VMEM((CHUNK,), jnp.int32),
            pltpu.VMEM((CHUNK,), jnp.int32),
            pltpu.VMEM((CHUNK,), jnp.int32),
            pltpu.VMEM((TAIL,), jnp.int32),
            pltpu.VMEM_SHARED((N, D), jnp.float32),
            pltpu.SemaphoreType.DMA,
            pltpu.SemaphoreType.DMA,
            pltpu.SemaphoreType.DMA,
            pltpu.SemaphoreType.DMA,
            pltpu.SemaphoreType.DMA,
            pltpu.SemaphoreType.DMA,
        ],
    )
    def k(m_hbm, d_hbm, out_hbm, mb0, mb1, mb2, ib0, ib1, ib2, itail,
          acc, ms0, ms1, ms2, is0, is1, is2):
        c = lax.axis_index("c")
        s = lax.axis_index("s")
        mbs = (mb0, mb1, mb2)
        ibs = (ib0, ib1, ib2)
        msems = (ms0, ms1, ms2)
        isems = (is0, is1, is2)

        ebase = c * (NS * EPT) + s * EPT

        def start_in(j, b):
            off = ebase + j * CHUNK
            pltpu.async_copy(d_hbm.at[pl.ds(off, CHUNK)], ibs[b], isems[b])
            pltpu.async_copy(m_hbm.at[pl.ds(off, CHUNK)], mbs[b], msems[b])

        def wait_in(b):
            pltpu.make_async_copy(
                d_hbm.at[pl.ds(0, CHUNK)], ibs[b], isems[b]).wait()
            pltpu.make_async_copy(
                m_hbm.at[pl.ds(0, CHUNK)], mbs[b], msems[b]).wait()

        start_in(0, 1)
        start_in(1, 2)

        zero = jnp.zeros((16,), jnp.float32)

        @pl.loop(0, ZCHUNK)
        def _(r):
            @pl.loop(0, D, step=16)
            def _(col):
                mb0[r, pl.ds(col, 16)] = zero

        @pl.loop(0, ZROWS // ZCHUNK)
        def _(j):
            pltpu.async_copy(
                mb0.at[pl.ds(0, ZCHUNK)],
                acc.at[pl.ds(s * ZROWS + j * ZCHUNK, ZCHUNK)],
                ms0,
            )

        @pl.when(s == NS - 1)
        def _():
            pltpu.async_copy(
                mb0.at[pl.ds(0, ZTAIL)],
                acc.at[pl.ds(NS * ZROWS, ZTAIL)],
                ms0,
            )

        @pl.loop(0, ZROWS // ZCHUNK)
        def _(j):
            pltpu.make_async_copy(
                mb0.at[pl.ds(0, ZCHUNK)],
                acc.at[pl.ds(0, ZCHUNK)],
                ms0,
            ).wait()

        @pl.when(s == NS - 1)
        def _():
            pltpu.make_async_copy(
                mb0.at[pl.ds(0, ZTAIL)],
                acc.at[pl.ds(0, ZTAIL)],
                ms0,
            ).wait()

        plsc.subcore_barrier()

        start_in(2, 0)

        RB = (1, 2, 0)

        @pl.loop(0, NFULL // NBUF)
        def _(it):
            j0 = it * NBUF
            hs = []
            for k in range(NBUF):
                b = RB[k]
                wait_in(b)
                hs.append(pltpu.async_copy(
                    mbs[b], acc.at[ibs[b]], msems[b], add=True))
            for k in range(NBUF):
                hs[k].wait()

                @pl.when(j0 + k + NBUF < NFULL)
                def _(k=k):
                    start_in(j0 + k + NBUF, RB[k])

        toff = ebase + NFULL * CHUNK
        pltpu.sync_copy(d_hbm.at[pl.ds(toff, TAIL)], itail)
        pltpu.sync_copy(m_hbm.at[pl.ds(toff, TAIL)], mb0.at[pl.ds(0, TAIL)])
        pltpu.sync_copy(mb0.at[pl.ds(0, TAIL)], acc.at[itail], add=True)

        plsc.subcore_barrier()

        @pl.loop(0, ZROWS // ZCHUNK)
        def _(j):
            row = s * ZROWS + j * ZCHUNK
            pltpu.async_copy(
                acc.at[pl.ds(row, ZCHUNK)],
                out_hbm.at[c].at[pl.ds(row, ZCHUNK)],
                ms0,
            )

        @pl.when(s == NS - 1)
        def _():
            pltpu.async_copy(
                acc.at[pl.ds(NS * ZROWS, ZTAIL)],
                out_hbm.at[c].at[pl.ds(NS * ZROWS, ZTAIL)],
                ms0,
            )

        @pl.loop(0, ZROWS // ZCHUNK)
        def _(j):
            pltpu.make_async_copy(
                acc.at[pl.ds(0, ZCHUNK)],
                out_hbm.at[c].at[pl.ds(0, ZCHUNK)],
                ms0,
            ).wait()

        @pl.when(s == NS - 1)
        def _():
            pltpu.make_async_copy(
                acc.at[pl.ds(0, ZTAIL)],
                out_hbm.at[c].at[pl.ds(0, ZTAIL)],
                ms0,
            ).wait()

    return k(M, dest)


def _tc_add_kernel(a_ref, b_ref, o_ref):
    o_ref[...] = a_ref[0] + b_ref[0]


def _tc_add(partials):
    blk = 1000
    return pl.pallas_call(
        _tc_add_kernel,
        out_shape=jax.ShapeDtypeStruct((N, D), jnp.float32),
        grid=(N // blk,),
        in_specs=[
            pl.BlockSpec((1, blk, D), lambda i: (0, i, 0)),
            pl.BlockSpec((1, blk, D), lambda i: (1, i, 0)),
        ],
        out_specs=pl.BlockSpec((blk, D), lambda i: (i, 0)),
    )(partials, partials)


def kernel(M, dest, dim_size):
    partials = _sc_segment_sum(M, dest.astype(jnp.int32))
    out = _tc_add(partials)
    w = jnp.ones((E, 1), dtype=M.dtype)
    return (out, w)
